# Initial kernel scaffold; baseline (speedup 1.0000x reference)
import jax
import jax.numpy as jnp
from jax.experimental import pallas as pl


def kernel(scores, rois, img_size):
    raise NotImplementedError("write your pallas kernel here")


# speedup vs baseline: 368.3181x; 368.3181x over previous
import functools

import jax
import jax.numpy as jnp
from jax import lax
from jax.experimental import pallas as pl
from jax.experimental.pallas import tpu as pltpu
from jax.experimental.pallas import tpu_sc as plsc

B = 4
N = 5000
NPAD = 5008
NCHUNKS = NPAD // 16
POST = 300
KPAD = 304
KCHUNKS = KPAD // 16
THRESH = 0.7

NC = 2
NS = 16
NW = NC * NS
NCLIP = NW - B
ROWS_FULL = 720
ROWS_LAST = B * N - ROWS_FULL * (NCLIP - 1)


def _nms_body(b, rois_hbm, order_hbm, idx_hbm, roi_v, ord_v,
              x1v, y1v, x2v, y2v, arv, kx1, ky1, kx2, ky2, kar, oidx,
              mx, my):
    pltpu.sync_copy(rois_hbm.at[pl.ds(b * N * 4, N * 4)],
                    roi_v.at[pl.ds(0, N * 4)])
    pltpu.sync_copy(order_hbm.at[b], ord_v)

    z16 = jnp.zeros((16,), jnp.int32)
    zf = jnp.zeros((16,), jnp.float32)

    def prep(k, _):
        base = k * 16
        ordc = ord_v[pl.ds(base, 16)] * 4
        cx = plsc.load_gather(roi_v, [ordc])
        cy = plsc.load_gather(roi_v, [ordc + 1])
        w = plsc.load_gather(roi_v, [ordc + 2])
        h = plsc.load_gather(roi_v, [ordc + 3])
        x1 = jnp.minimum(jnp.maximum(cx - 0.5 * w, 0.0), mx)
        y1 = jnp.minimum(jnp.maximum(cy - 0.5 * h, 0.0), my)
        x2 = jnp.minimum(jnp.maximum(cx + 0.5 * w, 0.0), mx)
        y2 = jnp.minimum(jnp.maximum(cy + 0.5 * h, 0.0), my)
        x1v[pl.ds(base, 16)] = x1
        y1v[pl.ds(base, 16)] = y1
        x2v[pl.ds(base, 16)] = x2
        y2v[pl.ds(base, 16)] = y2
        arv[pl.ds(base, 16)] = (x2 - x1) * (y2 - y1)
        return 0

    lax.fori_loop(0, NCHUNKS, prep, 0)

    def init_kept(k, _):
        base = k * 16
        kx1[pl.ds(base, 16)] = zf
        ky1[pl.ds(base, 16)] = zf
        kx2[pl.ds(base, 16)] = zf
        ky2[pl.ds(base, 16)] = zf
        kar[pl.ds(base, 16)] = zf
        oidx[pl.ds(base, 16)] = z16 - 1
        return 0

    lax.fori_loop(0, KCHUNKS, init_kept, 0)

    lane0 = lax.iota(jnp.int32, 16) == 0

    def cond(state):
        i, cnt = state
        return (i < N) & (cnt < POST)

    def body(state):
        i, cnt = state
        i16 = z16 + i
        x1i = plsc.load_gather(x1v, [i16])
        y1i = plsc.load_gather(y1v, [i16])
        x2i = plsc.load_gather(x2v, [i16])
        y2i = plsc.load_gather(y2v, [i16])
        ari = plsc.load_gather(arv, [i16])

        def chk(c, acc):
            kb = c * 16
            a1 = kx1[pl.ds(kb, 16)]
            b1 = ky1[pl.ds(kb, 16)]
            a2 = kx2[pl.ds(kb, 16)]
            b2 = ky2[pl.ds(kb, 16)]
            ka = kar[pl.ds(kb, 16)]
            ww = jnp.maximum(jnp.minimum(x2i, a2) - jnp.maximum(x1i, a1), 0.0)
            hh = jnp.maximum(jnp.minimum(y2i, b2) - jnp.maximum(y1i, b1), 0.0)
            inter = ww * hh
            d = (ari + ka) - inter + 1e-9
            return acc | (inter > THRESH * d)

        nch = (cnt + 15) // 16
        hit = lax.fori_loop(0, nch, chk, jnp.zeros((16,), jnp.bool_))
        sup = jnp.any(hit)
        wm = lane0 & jnp.logical_not(sup)
        c16 = z16 + cnt
        plsc.store_scatter(kx1, [c16], x1i, mask=wm)
        plsc.store_scatter(ky1, [c16], y1i, mask=wm)
        plsc.store_scatter(kx2, [c16], x2i, mask=wm)
        plsc.store_scatter(ky2, [c16], y2i, mask=wm)
        plsc.store_scatter(kar, [c16], ari, mask=wm)
        ov = plsc.load_gather(ord_v, [i16])
        plsc.store_scatter(oidx, [c16], ov, mask=wm)
        return i + 1, cnt + jnp.where(sup, 0, 1)

    lax.while_loop(cond, body, (jnp.int32(0), jnp.int32(0)))
    pltpu.sync_copy(oidx, idx_hbm.at[b])


def _clip_rows(start, nrows, rois_hbm, clip_hbm, roi_v, clipout_v, mx, my):
    pltpu.sync_copy(rois_hbm.at[pl.ds(start * 4, nrows * 4)],
                    roi_v.at[pl.ds(0, nrows * 4)])
    li = lax.iota(jnp.int32, 16)

    def one(k, _):
        bidx = (li + k * 16) * 4
        cx = plsc.load_gather(roi_v, [bidx])
        cy = plsc.load_gather(roi_v, [bidx + 1])
        w = plsc.load_gather(roi_v, [bidx + 2])
        h = plsc.load_gather(roi_v, [bidx + 3])
        x1 = jnp.minimum(jnp.maximum(cx - 0.5 * w, 0.0), mx)
        y1 = jnp.minimum(jnp.maximum(cy - 0.5 * h, 0.0), my)
        x2 = jnp.minimum(jnp.maximum(cx + 0.5 * w, 0.0), mx)
        y2 = jnp.minimum(jnp.maximum(cy + 0.5 * h, 0.0), my)
        plsc.store_scatter(clipout_v, [bidx], x1)
        plsc.store_scatter(clipout_v, [bidx + 1], y1)
        plsc.store_scatter(clipout_v, [bidx + 2], x2)
        plsc.store_scatter(clipout_v, [bidx + 3], y2)
        return 0

    lax.fori_loop(0, nrows // 16, one, 0)
    pltpu.sync_copy(clipout_v.at[pl.ds(0, nrows * 4)],
                    clip_hbm.at[pl.ds(start * 4, nrows * 4)])


def _sc_kernel(rois_hbm, order_hbm, maxx_hbm, maxy_hbm, clip_hbm, idx_hbm,
               roi_v, ord_v, x1v, y1v, x2v, y2v, arv,
               kx1, ky1, kx2, ky2, kar, oidx, mx_v, my_v, clipout_v):
    wid = lax.axis_index("s") * NC + lax.axis_index("c")
    pltpu.sync_copy(maxx_hbm, mx_v)
    pltpu.sync_copy(maxy_hbm, my_v)
    mx = mx_v[...]
    my = my_v[...]

    @pl.when(wid < B)
    def _():
        _nms_body(wid, rois_hbm, order_hbm, idx_hbm, roi_v, ord_v,
                  x1v, y1v, x2v, y2v, arv, kx1, ky1, kx2, ky2, kar, oidx,
                  mx, my)

    t = wid - B

    @pl.when((wid >= B) & (t < NCLIP - 1))
    def _():
        _clip_rows(t * ROWS_FULL, ROWS_FULL, rois_hbm, clip_hbm,
                   roi_v, clipout_v, mx, my)

    @pl.when(t == NCLIP - 1)
    def _():
        _clip_rows(t * ROWS_FULL, ROWS_LAST, rois_hbm, clip_hbm,
                   roi_v, clipout_v, mx, my)


@jax.jit
def kernel(scores, rois, img_size):
    order = jnp.argsort(-scores, axis=-1).astype(jnp.int32)
    order = jnp.pad(order, ((0, 0), (0, NPAD - N)))
    rois1d = rois.reshape(B * N * 4)
    maxx16 = jnp.full((16,), img_size[1].astype(jnp.float32) - 1.0)
    maxy16 = jnp.full((16,), img_size[0].astype(jnp.float32) - 1.0)

    mesh = plsc.VectorSubcoreMesh(core_axis_name="c", subcore_axis_name="s",
                                  num_cores=NC, num_subcores=NS)
    run = pl.kernel(
        _sc_kernel,
        out_type=(jax.ShapeDtypeStruct((B * N * 4,), jnp.float32),
                  jax.ShapeDtypeStruct((B, KPAD), jnp.int32)),
        mesh=mesh,
        compiler_params=pltpu.CompilerParams(needs_layout_passes=False),
        scratch_types=[
            pltpu.VMEM((NPAD * 4,), jnp.float32),
            pltpu.VMEM((NPAD,), jnp.int32),
            pltpu.VMEM((NPAD,), jnp.float32),
            pltpu.VMEM((NPAD,), jnp.float32),
            pltpu.VMEM((NPAD,), jnp.float32),
            pltpu.VMEM((NPAD,), jnp.float32),
            pltpu.VMEM((NPAD,), jnp.float32),
            pltpu.VMEM((KPAD,), jnp.float32),
            pltpu.VMEM((KPAD,), jnp.float32),
            pltpu.VMEM((KPAD,), jnp.float32),
            pltpu.VMEM((KPAD,), jnp.float32),
            pltpu.VMEM((KPAD,), jnp.float32),
            pltpu.VMEM((KPAD,), jnp.int32),
            pltpu.VMEM((16,), jnp.float32),
            pltpu.VMEM((16,), jnp.float32),
            pltpu.VMEM((ROWS_FULL * 4,), jnp.float32),
        ],
    )
    clip1d, idxp = run(rois1d, order, maxx16, maxy16)
    return clip1d.reshape(B, N, 4), idxp[:, :POST]


# speedup vs baseline: 373.7136x; 1.0146x over previous
import functools

import jax
import jax.numpy as jnp
from jax import lax
from jax.experimental import pallas as pl
from jax.experimental.pallas import tpu as pltpu
from jax.experimental.pallas import tpu_sc as plsc

B = 4
N = 5000
NPAD = 5008
NCHUNKS = NPAD // 16
POST = 300
KPAD = 320
KCHUNKS = KPAD // 16
THRESH = 0.7

NC = 2
NS = 16
NW = NC * NS
NCLIP = NW - B
ROWS_FULL = 720
ROWS_LAST = B * N - ROWS_FULL * (NCLIP - 1)


def _nms_body(b, rois_hbm, order_hbm, idx_hbm, roi_v, ord_v,
              x1v, y1v, x2v, y2v, arv, kx1, ky1, kx2, ky2, kar, oidx,
              mx, my):
    pltpu.sync_copy(rois_hbm.at[pl.ds(b * N * 4, N * 4)],
                    roi_v.at[pl.ds(0, N * 4)])
    pltpu.sync_copy(order_hbm.at[b], ord_v)

    z16 = jnp.zeros((16,), jnp.int32)
    zf = jnp.zeros((16,), jnp.float32)

    def prep(k, _):
        base = k * 16
        ordc = ord_v[pl.ds(base, 16)] * 4
        cx = plsc.load_gather(roi_v, [ordc])
        cy = plsc.load_gather(roi_v, [ordc + 1])
        w = plsc.load_gather(roi_v, [ordc + 2])
        h = plsc.load_gather(roi_v, [ordc + 3])
        x1 = jnp.minimum(jnp.maximum(cx - 0.5 * w, 0.0), mx)
        y1 = jnp.minimum(jnp.maximum(cy - 0.5 * h, 0.0), my)
        x2 = jnp.minimum(jnp.maximum(cx + 0.5 * w, 0.0), mx)
        y2 = jnp.minimum(jnp.maximum(cy + 0.5 * h, 0.0), my)
        x1v[pl.ds(base, 16)] = x1
        y1v[pl.ds(base, 16)] = y1
        x2v[pl.ds(base, 16)] = x2
        y2v[pl.ds(base, 16)] = y2
        arv[pl.ds(base, 16)] = (x2 - x1) * (y2 - y1)
        return 0

    lax.fori_loop(0, NCHUNKS, prep, 0)

    def init_kept(k, _):
        base = k * 16
        kx1[pl.ds(base, 16)] = zf
        ky1[pl.ds(base, 16)] = zf
        kx2[pl.ds(base, 16)] = zf
        ky2[pl.ds(base, 16)] = zf
        kar[pl.ds(base, 16)] = zf
        oidx[pl.ds(base, 16)] = z16 - 1
        return 0

    lax.fori_loop(0, KCHUNKS, init_kept, 0)

    lane0 = lax.iota(jnp.int32, 16) == 0

    def cond(state):
        i, cnt = state
        return (i < N) & (cnt < POST)

    def body(state):
        i, cnt = state
        i16 = z16 + i
        x1i = plsc.load_gather(x1v, [i16])
        y1i = plsc.load_gather(y1v, [i16])
        x2i = plsc.load_gather(x2v, [i16])
        y2i = plsc.load_gather(y2v, [i16])
        ari = plsc.load_gather(arv, [i16])

        def iou_hit(kb):
            a1 = kx1[pl.ds(kb, 16)]
            b1 = ky1[pl.ds(kb, 16)]
            a2 = kx2[pl.ds(kb, 16)]
            b2 = ky2[pl.ds(kb, 16)]
            ka = kar[pl.ds(kb, 16)]
            ww = jnp.maximum(jnp.minimum(x2i, a2) - jnp.maximum(x1i, a1), 0.0)
            hh = jnp.maximum(jnp.minimum(y2i, b2) - jnp.maximum(y1i, b1), 0.0)
            inter = ww * hh
            d = (ari + ka) - inter + 1e-9
            return inter > THRESH * d

        def chk(c, acc):
            kb = c * 32
            return acc | iou_hit(kb) | iou_hit(kb + 16)

        nch = (cnt + 31) // 32
        hit = lax.fori_loop(0, nch, chk, jnp.zeros((16,), jnp.bool_))
        sup = jnp.any(hit)
        wm = lane0 & jnp.logical_not(sup)
        c16 = z16 + cnt
        plsc.store_scatter(kx1, [c16], x1i, mask=wm)
        plsc.store_scatter(ky1, [c16], y1i, mask=wm)
        plsc.store_scatter(kx2, [c16], x2i, mask=wm)
        plsc.store_scatter(ky2, [c16], y2i, mask=wm)
        plsc.store_scatter(kar, [c16], ari, mask=wm)
        ov = plsc.load_gather(ord_v, [i16])
        plsc.store_scatter(oidx, [c16], ov, mask=wm)
        return i + 1, cnt + jnp.where(sup, 0, 1)

    lax.while_loop(cond, body, (jnp.int32(0), jnp.int32(0)))
    pltpu.sync_copy(oidx, idx_hbm.at[b])


def _clip_rows(start, nrows, rois_hbm, clip_hbm, roi_v, clipout_v, mx, my):
    pltpu.sync_copy(rois_hbm.at[pl.ds(start * 4, nrows * 4)],
                    roi_v.at[pl.ds(0, nrows * 4)])
    li = lax.iota(jnp.int32, 16)

    def one(k, _):
        bidx = (li + k * 16) * 4
        cx = plsc.load_gather(roi_v, [bidx])
        cy = plsc.load_gather(roi_v, [bidx + 1])
        w = plsc.load_gather(roi_v, [bidx + 2])
        h = plsc.load_gather(roi_v, [bidx + 3])
        x1 = jnp.minimum(jnp.maximum(cx - 0.5 * w, 0.0), mx)
        y1 = jnp.minimum(jnp.maximum(cy - 0.5 * h, 0.0), my)
        x2 = jnp.minimum(jnp.maximum(cx + 0.5 * w, 0.0), mx)
        y2 = jnp.minimum(jnp.maximum(cy + 0.5 * h, 0.0), my)
        plsc.store_scatter(clipout_v, [bidx], x1)
        plsc.store_scatter(clipout_v, [bidx + 1], y1)
        plsc.store_scatter(clipout_v, [bidx + 2], x2)
        plsc.store_scatter(clipout_v, [bidx + 3], y2)
        return 0

    lax.fori_loop(0, nrows // 16, one, 0)
    pltpu.sync_copy(clipout_v.at[pl.ds(0, nrows * 4)],
                    clip_hbm.at[pl.ds(start * 4, nrows * 4)])


def _sc_kernel(rois_hbm, order_hbm, maxx_hbm, maxy_hbm, clip_hbm, idx_hbm,
               roi_v, ord_v, x1v, y1v, x2v, y2v, arv,
               kx1, ky1, kx2, ky2, kar, oidx, mx_v, my_v, clipout_v):
    wid = lax.axis_index("s") * NC + lax.axis_index("c")
    pltpu.sync_copy(maxx_hbm, mx_v)
    pltpu.sync_copy(maxy_hbm, my_v)
    mx = mx_v[...]
    my = my_v[...]

    @pl.when(wid < B)
    def _():
        _nms_body(wid, rois_hbm, order_hbm, idx_hbm, roi_v, ord_v,
                  x1v, y1v, x2v, y2v, arv, kx1, ky1, kx2, ky2, kar, oidx,
                  mx, my)

    t = wid - B

    @pl.when((wid >= B) & (t < NCLIP - 1))
    def _():
        _clip_rows(t * ROWS_FULL, ROWS_FULL, rois_hbm, clip_hbm,
                   roi_v, clipout_v, mx, my)

    @pl.when(t == NCLIP - 1)
    def _():
        _clip_rows(t * ROWS_FULL, ROWS_LAST, rois_hbm, clip_hbm,
                   roi_v, clipout_v, mx, my)


@jax.jit
def kernel(scores, rois, img_size):
    order = jnp.argsort(-scores, axis=-1).astype(jnp.int32)
    order = jnp.pad(order, ((0, 0), (0, NPAD - N)))
    rois1d = rois.reshape(B * N * 4)
    maxx16 = jnp.full((16,), img_size[1].astype(jnp.float32) - 1.0)
    maxy16 = jnp.full((16,), img_size[0].astype(jnp.float32) - 1.0)

    mesh = plsc.VectorSubcoreMesh(core_axis_name="c", subcore_axis_name="s",
                                  num_cores=NC, num_subcores=NS)
    run = pl.kernel(
        _sc_kernel,
        out_type=(jax.ShapeDtypeStruct((B * N * 4,), jnp.float32),
                  jax.ShapeDtypeStruct((B, KPAD), jnp.int32)),
        mesh=mesh,
        compiler_params=pltpu.CompilerParams(needs_layout_passes=False),
        scratch_types=[
            pltpu.VMEM((NPAD * 4,), jnp.float32),
            pltpu.VMEM((NPAD,), jnp.int32),
            pltpu.VMEM((NPAD,), jnp.float32),
            pltpu.VMEM((NPAD,), jnp.float32),
            pltpu.VMEM((NPAD,), jnp.float32),
            pltpu.VMEM((NPAD,), jnp.float32),
            pltpu.VMEM((NPAD,), jnp.float32),
            pltpu.VMEM((KPAD,), jnp.float32),
            pltpu.VMEM((KPAD,), jnp.float32),
            pltpu.VMEM((KPAD,), jnp.float32),
            pltpu.VMEM((KPAD,), jnp.float32),
            pltpu.VMEM((KPAD,), jnp.float32),
            pltpu.VMEM((KPAD,), jnp.int32),
            pltpu.VMEM((16,), jnp.float32),
            pltpu.VMEM((16,), jnp.float32),
            pltpu.VMEM((ROWS_FULL * 4,), jnp.float32),
        ],
    )
    clip1d, idxp = run(rois1d, order, maxx16, maxy16)
    return clip1d.reshape(B, N, 4), idxp[:, :POST]


# speedup vs baseline: 378.0898x; 1.0117x over previous
import functools

import jax
import jax.numpy as jnp
from jax import lax
from jax.experimental import pallas as pl
from jax.experimental.pallas import tpu as pltpu
from jax.experimental.pallas import tpu_sc as plsc

B = 4
N = 5000
NPAD = 5008
NCHUNKS = NPAD // 16
POST = 300
KPAD = 320
KCHUNKS = KPAD // 16
THRESH = 0.7

NC = 1
NS = 16
NW = NC * NS
NCLIP = NW - B
ROWS_FULL = 1680
ROWS_LAST = B * N - ROWS_FULL * (NCLIP - 1)


def _nms_body(b, rois_hbm, order_hbm, idx_hbm, roi_v, ord_v,
              x1v, y1v, x2v, y2v, arv, kx1, ky1, kx2, ky2, kar, oidx,
              mx, my):
    pltpu.sync_copy(rois_hbm.at[pl.ds(b * N * 4, N * 4)],
                    roi_v.at[pl.ds(0, N * 4)])
    pltpu.sync_copy(order_hbm.at[b], ord_v)

    z16 = jnp.zeros((16,), jnp.int32)
    zf = jnp.zeros((16,), jnp.float32)

    def prep(k, _):
        base = k * 16
        ordc = ord_v[pl.ds(base, 16)] * 4
        cx = plsc.load_gather(roi_v, [ordc])
        cy = plsc.load_gather(roi_v, [ordc + 1])
        w = plsc.load_gather(roi_v, [ordc + 2])
        h = plsc.load_gather(roi_v, [ordc + 3])
        x1 = jnp.minimum(jnp.maximum(cx - 0.5 * w, 0.0), mx)
        y1 = jnp.minimum(jnp.maximum(cy - 0.5 * h, 0.0), my)
        x2 = jnp.minimum(jnp.maximum(cx + 0.5 * w, 0.0), mx)
        y2 = jnp.minimum(jnp.maximum(cy + 0.5 * h, 0.0), my)
        x1v[pl.ds(base, 16)] = x1
        y1v[pl.ds(base, 16)] = y1
        x2v[pl.ds(base, 16)] = x2
        y2v[pl.ds(base, 16)] = y2
        arv[pl.ds(base, 16)] = (x2 - x1) * (y2 - y1)
        return 0

    lax.fori_loop(0, NCHUNKS, prep, 0)

    def init_kept(k, _):
        base = k * 16
        kx1[pl.ds(base, 16)] = zf
        ky1[pl.ds(base, 16)] = zf
        kx2[pl.ds(base, 16)] = zf
        ky2[pl.ds(base, 16)] = zf
        kar[pl.ds(base, 16)] = zf
        oidx[pl.ds(base, 16)] = z16 - 1
        return 0

    lax.fori_loop(0, KCHUNKS, init_kept, 0)

    lane0 = lax.iota(jnp.int32, 16) == 0

    def cond(state):
        i, cnt = state
        return (i < N) & (cnt < POST)

    def body(state):
        i, cnt = state
        i16 = z16 + i
        x1i = plsc.load_gather(x1v, [i16])
        y1i = plsc.load_gather(y1v, [i16])
        x2i = plsc.load_gather(x2v, [i16])
        y2i = plsc.load_gather(y2v, [i16])
        ari = plsc.load_gather(arv, [i16])

        def iou_hit(kb):
            a1 = kx1[pl.ds(kb, 16)]
            b1 = ky1[pl.ds(kb, 16)]
            a2 = kx2[pl.ds(kb, 16)]
            b2 = ky2[pl.ds(kb, 16)]
            ka = kar[pl.ds(kb, 16)]
            ww = jnp.maximum(jnp.minimum(x2i, a2) - jnp.maximum(x1i, a1), 0.0)
            hh = jnp.maximum(jnp.minimum(y2i, b2) - jnp.maximum(y1i, b1), 0.0)
            inter = ww * hh
            d = (ari + ka) - inter + 1e-9
            return inter > THRESH * d

        def chk(c, acc):
            kb = c * 32
            return acc | iou_hit(kb) | iou_hit(kb + 16)

        nch = (cnt + 31) // 32
        hit = lax.fori_loop(0, nch, chk, jnp.zeros((16,), jnp.bool_))
        sup = jnp.any(hit)
        wm = lane0 & jnp.logical_not(sup)
        c16 = z16 + cnt
        plsc.store_scatter(kx1, [c16], x1i, mask=wm)
        plsc.store_scatter(ky1, [c16], y1i, mask=wm)
        plsc.store_scatter(kx2, [c16], x2i, mask=wm)
        plsc.store_scatter(ky2, [c16], y2i, mask=wm)
        plsc.store_scatter(kar, [c16], ari, mask=wm)
        ov = plsc.load_gather(ord_v, [i16])
        plsc.store_scatter(oidx, [c16], ov, mask=wm)
        return i + 1, cnt + jnp.where(sup, 0, 1)

    lax.while_loop(cond, body, (jnp.int32(0), jnp.int32(0)))
    pltpu.sync_copy(oidx, idx_hbm.at[b])


def _clip_rows(start, nrows, rois_hbm, clip_hbm, roi_v, clipout_v, mx, my):
    pltpu.sync_copy(rois_hbm.at[pl.ds(start * 4, nrows * 4)],
                    roi_v.at[pl.ds(0, nrows * 4)])
    li = lax.iota(jnp.int32, 16)

    def one(k, _):
        bidx = (li + k * 16) * 4
        cx = plsc.load_gather(roi_v, [bidx])
        cy = plsc.load_gather(roi_v, [bidx + 1])
        w = plsc.load_gather(roi_v, [bidx + 2])
        h = plsc.load_gather(roi_v, [bidx + 3])
        x1 = jnp.minimum(jnp.maximum(cx - 0.5 * w, 0.0), mx)
        y1 = jnp.minimum(jnp.maximum(cy - 0.5 * h, 0.0), my)
        x2 = jnp.minimum(jnp.maximum(cx + 0.5 * w, 0.0), mx)
        y2 = jnp.minimum(jnp.maximum(cy + 0.5 * h, 0.0), my)
        plsc.store_scatter(clipout_v, [bidx], x1)
        plsc.store_scatter(clipout_v, [bidx + 1], y1)
        plsc.store_scatter(clipout_v, [bidx + 2], x2)
        plsc.store_scatter(clipout_v, [bidx + 3], y2)
        return 0

    lax.fori_loop(0, nrows // 16, one, 0)
    pltpu.sync_copy(clipout_v.at[pl.ds(0, nrows * 4)],
                    clip_hbm.at[pl.ds(start * 4, nrows * 4)])


def _sc_kernel(rois_hbm, order_hbm, maxx_hbm, maxy_hbm, clip_hbm, idx_hbm,
               roi_v, ord_v, x1v, y1v, x2v, y2v, arv,
               kx1, ky1, kx2, ky2, kar, oidx, mx_v, my_v, clipout_v):
    wid = lax.axis_index("s") * NC + lax.axis_index("c")
    pltpu.sync_copy(maxx_hbm, mx_v)
    pltpu.sync_copy(maxy_hbm, my_v)
    mx = mx_v[...]
    my = my_v[...]

    @pl.when(wid < B)
    def _():
        _nms_body(wid, rois_hbm, order_hbm, idx_hbm, roi_v, ord_v,
                  x1v, y1v, x2v, y2v, arv, kx1, ky1, kx2, ky2, kar, oidx,
                  mx, my)

    t = wid - B

    @pl.when((wid >= B) & (t < NCLIP - 1))
    def _():
        _clip_rows(t * ROWS_FULL, ROWS_FULL, rois_hbm, clip_hbm,
                   roi_v, clipout_v, mx, my)

    @pl.when(t == NCLIP - 1)
    def _():
        _clip_rows(t * ROWS_FULL, ROWS_LAST, rois_hbm, clip_hbm,
                   roi_v, clipout_v, mx, my)


@jax.jit
def kernel(scores, rois, img_size):
    order = jnp.argsort(-scores, axis=-1).astype(jnp.int32)
    order = jnp.pad(order, ((0, 0), (0, NPAD - N)))
    rois1d = rois.reshape(B * N * 4)
    maxx16 = jnp.full((16,), img_size[1].astype(jnp.float32) - 1.0)
    maxy16 = jnp.full((16,), img_size[0].astype(jnp.float32) - 1.0)

    mesh = plsc.VectorSubcoreMesh(core_axis_name="c", subcore_axis_name="s",
                                  num_cores=NC, num_subcores=NS)
    run = pl.kernel(
        _sc_kernel,
        out_type=(jax.ShapeDtypeStruct((B * N * 4,), jnp.float32),
                  jax.ShapeDtypeStruct((B, KPAD), jnp.int32)),
        mesh=mesh,
        compiler_params=pltpu.CompilerParams(needs_layout_passes=False),
        scratch_types=[
            pltpu.VMEM((NPAD * 4,), jnp.float32),
            pltpu.VMEM((NPAD,), jnp.int32),
            pltpu.VMEM((NPAD,), jnp.float32),
            pltpu.VMEM((NPAD,), jnp.float32),
            pltpu.VMEM((NPAD,), jnp.float32),
            pltpu.VMEM((NPAD,), jnp.float32),
            pltpu.VMEM((NPAD,), jnp.float32),
            pltpu.VMEM((KPAD,), jnp.float32),
            pltpu.VMEM((KPAD,), jnp.float32),
            pltpu.VMEM((KPAD,), jnp.float32),
            pltpu.VMEM((KPAD,), jnp.float32),
            pltpu.VMEM((KPAD,), jnp.float32),
            pltpu.VMEM((KPAD,), jnp.int32),
            pltpu.VMEM((16,), jnp.float32),
            pltpu.VMEM((16,), jnp.float32),
            pltpu.VMEM((ROWS_FULL * 4,), jnp.float32),
        ],
    )
    clip1d, idxp = run(rois1d, order, maxx16, maxy16)
    return clip1d.reshape(B, N, 4), idxp[:, :POST]


# speedup vs baseline: 378.1854x; 1.0003x over previous
import functools

import jax
import jax.numpy as jnp
from jax import lax
from jax.experimental import pallas as pl
from jax.experimental.pallas import tpu as pltpu
from jax.experimental.pallas import tpu_sc as plsc

B = 4
N = 5000
NPAD = 5008
NCHUNKS = NPAD // 16
POST = 300
KPAD = 320
KCHUNKS = KPAD // 16
THRESH = 0.7

NC = 1
NS = 16
NW = NC * NS
NCLIP = NW - B
ROWS_FULL = 1680
ROWS_LAST = B * N - ROWS_FULL * (NCLIP - 1)


def _nms_body(b, rois_hbm, order_hbm, idx_hbm, roi_v, ord_v,
              x1v, y1v, x2v, y2v, arv, kx1, ky1, kx2, ky2, kar, oidx,
              mx, my):
    pltpu.sync_copy(rois_hbm.at[pl.ds(b * N * 4, N * 4)],
                    roi_v.at[pl.ds(0, N * 4)])
    pltpu.sync_copy(order_hbm.at[b], ord_v)

    z16 = jnp.zeros((16,), jnp.int32)
    zf = jnp.zeros((16,), jnp.float32)

    def prep(k, _):
        base = k * 16
        ordc = ord_v[pl.ds(base, 16)] * 4
        cx = plsc.load_gather(roi_v, [ordc])
        cy = plsc.load_gather(roi_v, [ordc + 1])
        w = plsc.load_gather(roi_v, [ordc + 2])
        h = plsc.load_gather(roi_v, [ordc + 3])
        x1 = jnp.minimum(jnp.maximum(cx - 0.5 * w, 0.0), mx)
        y1 = jnp.minimum(jnp.maximum(cy - 0.5 * h, 0.0), my)
        x2 = jnp.minimum(jnp.maximum(cx + 0.5 * w, 0.0), mx)
        y2 = jnp.minimum(jnp.maximum(cy + 0.5 * h, 0.0), my)
        x1v[pl.ds(base, 16)] = x1
        y1v[pl.ds(base, 16)] = y1
        x2v[pl.ds(base, 16)] = x2
        y2v[pl.ds(base, 16)] = y2
        arv[pl.ds(base, 16)] = (x2 - x1) * (y2 - y1)
        return 0

    lax.fori_loop(0, NCHUNKS, prep, 0)

    def init_kept(k, _):
        base = k * 16
        kx1[pl.ds(base, 16)] = zf
        ky1[pl.ds(base, 16)] = zf
        kx2[pl.ds(base, 16)] = zf
        ky2[pl.ds(base, 16)] = zf
        kar[pl.ds(base, 16)] = zf
        oidx[pl.ds(base, 16)] = z16 - 1
        return 0

    lax.fori_loop(0, KCHUNKS, init_kept, 0)

    lane0 = lax.iota(jnp.int32, 16) == 0

    def cond(state):
        i, cnt = state
        return (i < N) & (cnt < POST)

    def body(state):
        i, cnt = state
        i16 = z16 + i
        x1i = plsc.load_gather(x1v, [i16])
        y1i = plsc.load_gather(y1v, [i16])
        x2i = plsc.load_gather(x2v, [i16])
        y2i = plsc.load_gather(y2v, [i16])
        ari = plsc.load_gather(arv, [i16])

        def iou_hit(kb):
            a1 = kx1[pl.ds(kb, 16)]
            b1 = ky1[pl.ds(kb, 16)]
            a2 = kx2[pl.ds(kb, 16)]
            b2 = ky2[pl.ds(kb, 16)]
            ka = kar[pl.ds(kb, 16)]
            ww = jnp.maximum(jnp.minimum(x2i, a2) - jnp.maximum(x1i, a1), 0.0)
            hh = jnp.maximum(jnp.minimum(y2i, b2) - jnp.maximum(y1i, b1), 0.0)
            inter = ww * hh
            d = (ari + ka) - inter + 1e-9
            return inter > THRESH * d

        def chk(c, acc):
            kb = c * 32
            return acc | iou_hit(kb) | iou_hit(kb + 16)

        nch = (cnt + 31) // 32
        hit = lax.fori_loop(0, nch, chk, jnp.zeros((16,), jnp.bool_))
        sup = jnp.any(hit)
        wm = lane0 & jnp.logical_not(sup)
        c16 = z16 + cnt
        plsc.store_scatter(kx1, [c16], x1i, mask=wm)
        plsc.store_scatter(ky1, [c16], y1i, mask=wm)
        plsc.store_scatter(kx2, [c16], x2i, mask=wm)
        plsc.store_scatter(ky2, [c16], y2i, mask=wm)
        plsc.store_scatter(kar, [c16], ari, mask=wm)
        ov = plsc.load_gather(ord_v, [i16])
        plsc.store_scatter(oidx, [c16], ov, mask=wm)
        return i + 1, cnt + jnp.where(sup, 0, 1)

    lax.while_loop(cond, body, (jnp.int32(0), jnp.int32(0)))
    pltpu.sync_copy(oidx, idx_hbm.at[b])


def _clip_rows(start, nrows, rois_hbm, clip_hbm, roi_v, clipout_v, mx, my):
    pltpu.sync_copy(rois_hbm.at[pl.ds(start * 4, nrows * 4)],
                    roi_v.at[pl.ds(0, nrows * 4)])
    li = lax.iota(jnp.int32, 16)

    def one(k, _):
        bidx = (li + k * 16) * 4
        cx = plsc.load_gather(roi_v, [bidx])
        cy = plsc.load_gather(roi_v, [bidx + 1])
        w = plsc.load_gather(roi_v, [bidx + 2])
        h = plsc.load_gather(roi_v, [bidx + 3])
        x1 = jnp.minimum(jnp.maximum(cx - 0.5 * w, 0.0), mx)
        y1 = jnp.minimum(jnp.maximum(cy - 0.5 * h, 0.0), my)
        x2 = jnp.minimum(jnp.maximum(cx + 0.5 * w, 0.0), mx)
        y2 = jnp.minimum(jnp.maximum(cy + 0.5 * h, 0.0), my)
        plsc.store_scatter(clipout_v, [bidx], x1)
        plsc.store_scatter(clipout_v, [bidx + 1], y1)
        plsc.store_scatter(clipout_v, [bidx + 2], x2)
        plsc.store_scatter(clipout_v, [bidx + 3], y2)
        return 0

    lax.fori_loop(0, nrows // 16, one, 0)
    pltpu.sync_copy(clipout_v.at[pl.ds(0, nrows * 4)],
                    clip_hbm.at[pl.ds(start * 4, nrows * 4)])


def _sc_kernel(rois_hbm, order_hbm, maxx_hbm, maxy_hbm, clip_hbm, idx_hbm,
               roi_v, ord_v, x1v, y1v, x2v, y2v, arv,
               kx1, ky1, kx2, ky2, kar, oidx, mx_v, my_v, clipout_v):
    wid = lax.axis_index("s") * NC + lax.axis_index("c")
    pltpu.sync_copy(maxx_hbm, mx_v)
    pltpu.sync_copy(maxy_hbm, my_v)
    mx = mx_v[...]
    my = my_v[...]

    @pl.when(wid < B)
    def _():
        _nms_body(wid, rois_hbm, order_hbm, idx_hbm, roi_v, ord_v,
                  x1v, y1v, x2v, y2v, arv, kx1, ky1, kx2, ky2, kar, oidx,
                  mx, my)

    t = wid - B

    @pl.when((wid >= B) & (t < NCLIP - 1))
    def _():
        _clip_rows(t * ROWS_FULL, ROWS_FULL, rois_hbm, clip_hbm,
                   roi_v, clipout_v, mx, my)

    @pl.when(t == NCLIP - 1)
    def _():
        _clip_rows(t * ROWS_FULL, ROWS_LAST, rois_hbm, clip_hbm,
                   roi_v, clipout_v, mx, my)


def _kernel_impl(scores, rois, img_size):
    order = jnp.argsort(-scores, axis=-1).astype(jnp.int32)
    order = jnp.pad(order, ((0, 0), (0, NPAD - N)))
    rois1d = rois.reshape(B * N * 4)
    maxx16 = jnp.full((16,), img_size[1].astype(jnp.float32) - 1.0)
    maxy16 = jnp.full((16,), img_size[0].astype(jnp.float32) - 1.0)

    mesh = plsc.VectorSubcoreMesh(core_axis_name="c", subcore_axis_name="s",
                                  num_cores=NC, num_subcores=NS)
    run = pl.kernel(
        _sc_kernel,
        out_type=(jax.ShapeDtypeStruct((B * N * 4,), jnp.float32),
                  jax.ShapeDtypeStruct((B, KPAD), jnp.int32)),
        mesh=mesh,
        compiler_params=pltpu.CompilerParams(needs_layout_passes=False),
        scratch_types=[
            pltpu.VMEM((NPAD * 4,), jnp.float32),
            pltpu.VMEM((NPAD,), jnp.int32),
            pltpu.VMEM((NPAD,), jnp.float32),
            pltpu.VMEM((NPAD,), jnp.float32),
            pltpu.VMEM((NPAD,), jnp.float32),
            pltpu.VMEM((NPAD,), jnp.float32),
            pltpu.VMEM((NPAD,), jnp.float32),
            pltpu.VMEM((KPAD,), jnp.float32),
            pltpu.VMEM((KPAD,), jnp.float32),
            pltpu.VMEM((KPAD,), jnp.float32),
            pltpu.VMEM((KPAD,), jnp.float32),
            pltpu.VMEM((KPAD,), jnp.float32),
            pltpu.VMEM((KPAD,), jnp.int32),
            pltpu.VMEM((16,), jnp.float32),
            pltpu.VMEM((16,), jnp.float32),
            pltpu.VMEM((ROWS_FULL * 4,), jnp.float32),
        ],
    )
    clip1d, idxp = run(rois1d, order, maxx16, maxy16)
    return clip1d.reshape(B, N, 4), idxp[:, :POST]


from jax.experimental import layout as _layout

_jit_cache = {}
_plain_jit = jax.jit(_kernel_impl)


def kernel(scores, rois, img_size):
    dev = getattr(scores, "device", None)
    if not isinstance(dev, jax.Device) or dev.platform != "tpu":
        return _plain_jit(scores, rois, img_size)
    fn = _jit_cache.get(dev)
    if fn is None:
        sh = jax.sharding.SingleDeviceSharding(dev)
        fn = jax.jit(
            _kernel_impl,
            out_shardings=(
                _layout.Format(_layout.Layout(major_to_minor=(0, 1, 2),
                                              tiling=()), sh),
                _layout.Format(_layout.Layout(major_to_minor=(0, 1),
                                              tiling=()), sh),
            ),
        )
        _jit_cache[dev] = fn
    return fn(scores, rois, img_size)
